# baseline (speedup 1.0000x reference)
import functools

import jax
import jax.numpy as jnp
from jax import lax
from jax.experimental import pallas as pl
from jax.experimental.pallas import tpu as pltpu
from jax.experimental.pallas import tpu_sc as plsc

B = 4096
L = 50
V = 100000
D = 128
H = 256
C = 2

NC = 2
NS = 16
NW = NC * NS
BAGS_PER_W = B // NW
CHUNK_BAGS = 2
CHUNK_IDX = CHUNK_BAGS * L
CHUNKS_PER_W = BAGS_PER_W // CHUNK_BAGS
G = D // 16


def _sc_bag_sums_body(table_hbm, tags_hbm, out_hbm, idx_v, rows0, rows1,
                      out_v, sem0, sem1):
    cid = lax.axis_index("c")
    sid = lax.axis_index("s")
    wid = sid * NC + cid
    bufs = (rows0, rows1)
    sems = (sem0, sem1)

    pltpu.sync_copy(tags_hbm.at[pl.ds(wid * CHUNKS_PER_W, CHUNKS_PER_W)], idx_v)

    for b in range(2):
        pltpu.async_copy(table_hbm.at[idx_v.at[b]], bufs[b], sems[b])

    def outer(cc, carry):
        for b in range(2):
            ci = 2 * cc + b
            pltpu.make_async_copy(
                table_hbm.at[idx_v.at[ci]], bufs[b], sems[b]).wait()
            for b2 in range(CHUNK_BAGS):
                base = b2 * L
                acc = [bufs[b][base, pl.ds(g * 16, 16)] for g in range(G)]
                for r in range(1, L):
                    for g in range(G):
                        acc[g] = acc[g] + bufs[b][base + r, pl.ds(g * 16, 16)]
                row = CHUNK_BAGS * ci + b2
                for g in range(G):
                    out_v[row, pl.ds(g * 16, 16)] = acc[g]
            nci = ci + 2

            @pl.when(nci < CHUNKS_PER_W)
            def _():
                pltpu.async_copy(table_hbm.at[idx_v.at[nci]], bufs[b], sems[b])
        return carry

    lax.fori_loop(0, CHUNKS_PER_W // 2, outer, 0)
    pltpu.sync_copy(out_v, out_hbm.at[pl.ds(wid * BAGS_PER_W, BAGS_PER_W)])


@jax.jit
def _sc_bag_sums(emb_weight, tags2d):
    mesh = plsc.VectorSubcoreMesh(core_axis_name="c", subcore_axis_name="s")
    return pl.kernel(
        _sc_bag_sums_body,
        out_type=jax.ShapeDtypeStruct((B, D), jnp.float32),
        mesh=mesh,
        scratch_types=[
            pltpu.VMEM((CHUNKS_PER_W, CHUNK_IDX), jnp.int32),
            pltpu.VMEM((CHUNK_IDX, D), jnp.float32),
            pltpu.VMEM((CHUNK_IDX, D), jnp.float32),
            pltpu.VMEM((BAGS_PER_W, D), jnp.float32),
            pltpu.SemaphoreType.DMA,
            pltpu.SemaphoreType.DMA,
        ],
    )(emb_weight, tags2d)


ROWS_BLK = 512


def _mlp_body(bags_ref, emos_ref, w1_ref, w2_ref, b1_ref, wo_ref, bo_ref,
              out_ref):
    feats = jnp.maximum(bags_ref[...] * (1.0 / L), 0.0)
    h = jnp.dot(feats, w1_ref[...].T, preferred_element_type=jnp.float32)
    h = h + jnp.dot(emos_ref[...], w2_ref[...].T,
                    preferred_element_type=jnp.float32)
    h = jnp.maximum(h + b1_ref[...], 0.0)
    out_ref[...] = (
        jnp.dot(h, wo_ref[...].T, preferred_element_type=jnp.float32)
        + bo_ref[...]
    )


@jax.jit
def _mlp(bag_sums, emos, w1, w2, b1, wo, bo):
    nblk = B // ROWS_BLK
    return pl.pallas_call(
        _mlp_body,
        out_shape=jax.ShapeDtypeStruct((B, C), jnp.float32),
        grid=(nblk,),
        in_specs=[
            pl.BlockSpec((ROWS_BLK, D), lambda i: (i, 0)),
            pl.BlockSpec((ROWS_BLK, 2), lambda i: (i, 0)),
            pl.BlockSpec((H, D), lambda i: (0, 0)),
            pl.BlockSpec((H, 2), lambda i: (0, 0)),
            pl.BlockSpec((1, H), lambda i: (0, 0)),
            pl.BlockSpec((C, H), lambda i: (0, 0)),
            pl.BlockSpec((1, C), lambda i: (0, 0)),
        ],
        out_specs=pl.BlockSpec((ROWS_BLK, C), lambda i: (i, 0)),
    )(bag_sums, emos, w1, w2, b1, wo, bo)


def kernel(emos, tags_vec, offsets, emb_weight, hid_w, hid_b, out_w, out_b):
    del offsets
    tags2d = tags_vec.reshape(NW * CHUNKS_PER_W, CHUNK_IDX)
    bag_sums = _sc_bag_sums(emb_weight, tags2d)
    w1 = hid_w[:, :D]
    w2 = hid_w[:, D:]
    return _mlp(bag_sums, emos, w1, w2, hid_b.reshape(1, H), out_w,
                out_b.reshape(1, C))


# speedup vs baseline: 1.8042x; 1.8042x over previous
import functools

import jax
import jax.numpy as jnp
from jax import lax
from jax.experimental import pallas as pl
from jax.experimental.pallas import tpu as pltpu
from jax.experimental.pallas import tpu_sc as plsc

B = 4096
L = 50
V = 100000
D = 128
H = 256
C = 2

NC = 2
NS = 16
NW = NC * NS
BAGS_PER_W = B // NW
CHUNK_BAGS = 2
CHUNK_IDX = CHUNK_BAGS * L
CHUNKS_PER_W = BAGS_PER_W // CHUNK_BAGS
G = D // 16


def _sc_bag_sums_body(table_hbm, tags_hbm, out_hbm, idx_v, rows0, rows1,
                      out_v, sem0, sem1):
    cid = lax.axis_index("c")
    sid = lax.axis_index("s")
    wid = sid * NC + cid
    bufs = (rows0, rows1)
    sems = (sem0, sem1)

    pltpu.sync_copy(tags_hbm.at[pl.ds(wid * CHUNKS_PER_W, CHUNKS_PER_W)], idx_v)

    for b in range(2):
        pltpu.async_copy(table_hbm.at[idx_v.at[b]], bufs[b], sems[b])

    def outer(cc, carry):
        for b in range(2):
            ci = 2 * cc + b
            pltpu.make_async_copy(
                table_hbm.at[idx_v.at[ci]], bufs[b], sems[b]).wait()
            for b2 in range(CHUNK_BAGS):
                base = b2 * L

                def accum(r, acc, _b=b, _base=base):
                    return tuple(
                        acc[g] + bufs[_b][_base + r, pl.ds(g * 16, 16)]
                        for g in range(G)
                    )

                zeros = tuple(jnp.zeros((16,), jnp.float32) for _ in range(G))
                acc = lax.fori_loop(0, L, accum, zeros)
                row = CHUNK_BAGS * ci + b2
                for g in range(G):
                    out_v[row, pl.ds(g * 16, 16)] = acc[g]
            nci = ci + 2

            @pl.when(nci < CHUNKS_PER_W)
            def _():
                pltpu.async_copy(table_hbm.at[idx_v.at[nci]], bufs[b], sems[b])
        return carry

    lax.fori_loop(0, CHUNKS_PER_W // 2, outer, 0)
    pltpu.sync_copy(out_v, out_hbm.at[pl.ds(wid * BAGS_PER_W, BAGS_PER_W)])


@jax.jit
def _sc_bag_sums(emb_weight, tags2d):
    mesh = plsc.VectorSubcoreMesh(core_axis_name="c", subcore_axis_name="s")
    return pl.kernel(
        _sc_bag_sums_body,
        out_type=jax.ShapeDtypeStruct((B, D), jnp.float32),
        mesh=mesh,
        scratch_types=[
            pltpu.VMEM((CHUNKS_PER_W, CHUNK_IDX), jnp.int32),
            pltpu.VMEM((CHUNK_IDX, D), jnp.float32),
            pltpu.VMEM((CHUNK_IDX, D), jnp.float32),
            pltpu.VMEM((BAGS_PER_W, D), jnp.float32),
            pltpu.SemaphoreType.DMA,
            pltpu.SemaphoreType.DMA,
        ],
    )(emb_weight, tags2d)


ROWS_BLK = 512


def _mlp_body(bags_ref, emos_ref, w1_ref, w2_ref, b1_ref, wo_ref, bo_ref,
              out_ref):
    feats = jnp.maximum(bags_ref[...] * (1.0 / L), 0.0)
    h = jnp.dot(feats, w1_ref[...].T, preferred_element_type=jnp.float32)
    h = h + jnp.dot(emos_ref[...], w2_ref[...].T,
                    preferred_element_type=jnp.float32)
    h = jnp.maximum(h + b1_ref[...], 0.0)
    out_ref[...] = (
        jnp.dot(h, wo_ref[...].T, preferred_element_type=jnp.float32)
        + bo_ref[...]
    )


@jax.jit
def _mlp(bag_sums, emos, w1, w2, b1, wo, bo):
    nblk = B // ROWS_BLK
    return pl.pallas_call(
        _mlp_body,
        out_shape=jax.ShapeDtypeStruct((B, C), jnp.float32),
        grid=(nblk,),
        in_specs=[
            pl.BlockSpec((ROWS_BLK, D), lambda i: (i, 0)),
            pl.BlockSpec((ROWS_BLK, 2), lambda i: (i, 0)),
            pl.BlockSpec((H, D), lambda i: (0, 0)),
            pl.BlockSpec((H, 2), lambda i: (0, 0)),
            pl.BlockSpec((1, H), lambda i: (0, 0)),
            pl.BlockSpec((C, H), lambda i: (0, 0)),
            pl.BlockSpec((1, C), lambda i: (0, 0)),
        ],
        out_specs=pl.BlockSpec((ROWS_BLK, C), lambda i: (i, 0)),
    )(bag_sums, emos, w1, w2, b1, wo, bo)


def kernel(emos, tags_vec, offsets, emb_weight, hid_w, hid_b, out_w, out_b):
    del offsets
    tags2d = tags_vec.reshape(NW * CHUNKS_PER_W, CHUNK_IDX)
    bag_sums = _sc_bag_sums(emb_weight, tags2d)
    w1 = hid_w[:, :D]
    w2 = hid_w[:, D:]
    return _mlp(bag_sums, emos, w1, w2, hid_b.reshape(1, H), out_w,
                out_b.reshape(1, C))


# speedup vs baseline: 1.8073x; 1.0017x over previous
import functools

import jax
import jax.numpy as jnp
from jax import lax
from jax.experimental import pallas as pl
from jax.experimental.pallas import tpu as pltpu
from jax.experimental.pallas import tpu_sc as plsc

B = 4096
L = 50
V = 100000
D = 128
H = 256
C = 2

NC = 2
NS = 16
NW = NC * NS
BAGS_PER_W = B // NW
CHUNK_BAGS = 2
CHUNK_IDX = CHUNK_BAGS * L
CHUNKS_PER_W = BAGS_PER_W // CHUNK_BAGS
G = D // 16


def _sc_bag_sums_body(table_hbm, tags_hbm, out_hbm, idx_v, rows0, rows1,
                      out_v, sem0, sem1):
    cid = lax.axis_index("c")
    sid = lax.axis_index("s")
    wid = sid * NC + cid
    bufs = (rows0, rows1)
    sems = (sem0, sem1)

    pltpu.sync_copy(tags_hbm.at[pl.ds(wid * CHUNKS_PER_W, CHUNKS_PER_W)], idx_v)

    for b in range(2):
        pltpu.async_copy(table_hbm.at[idx_v.at[b]], bufs[b], sems[b])

    def outer(cc, carry):
        for b in range(2):
            ci = 2 * cc + b
            pltpu.make_async_copy(
                table_hbm.at[idx_v.at[ci]], bufs[b], sems[b]).wait()
            for b2 in range(CHUNK_BAGS):
                base = b2 * L

                def accum(r, acc, _b=b, _base=base):
                    r2 = _base + r * 2
                    return tuple(
                        acc[g]
                        + bufs[_b][r2, pl.ds(g * 16, 16)]
                        + bufs[_b][r2 + 1, pl.ds(g * 16, 16)]
                        for g in range(G)
                    )

                zeros = tuple(jnp.zeros((16,), jnp.float32) for _ in range(G))
                acc = lax.fori_loop(0, L // 2, accum, zeros)
                row = CHUNK_BAGS * ci + b2
                for g in range(G):
                    out_v[row, pl.ds(g * 16, 16)] = acc[g]
            nci = ci + 2

            @pl.when(nci < CHUNKS_PER_W)
            def _():
                pltpu.async_copy(table_hbm.at[idx_v.at[nci]], bufs[b], sems[b])
        return carry

    lax.fori_loop(0, CHUNKS_PER_W // 2, outer, 0)
    pltpu.sync_copy(out_v, out_hbm.at[pl.ds(wid * BAGS_PER_W, BAGS_PER_W)])


@jax.jit
def _sc_bag_sums(emb_weight, tags2d):
    mesh = plsc.VectorSubcoreMesh(core_axis_name="c", subcore_axis_name="s")
    return pl.kernel(
        _sc_bag_sums_body,
        out_type=jax.ShapeDtypeStruct((B, D), jnp.float32),
        mesh=mesh,
        scratch_types=[
            pltpu.VMEM((CHUNKS_PER_W, CHUNK_IDX), jnp.int32),
            pltpu.VMEM((CHUNK_IDX, D), jnp.float32),
            pltpu.VMEM((CHUNK_IDX, D), jnp.float32),
            pltpu.VMEM((BAGS_PER_W, D), jnp.float32),
            pltpu.SemaphoreType.DMA,
            pltpu.SemaphoreType.DMA,
        ],
    )(emb_weight, tags2d)


ROWS_BLK = 512


def _mlp_body(bags_ref, emos_ref, w1_ref, w2_ref, b1_ref, wo_ref, bo_ref,
              out_ref):
    feats = jnp.maximum(bags_ref[...] * (1.0 / L), 0.0)
    h = jnp.dot(feats, w1_ref[...].T, preferred_element_type=jnp.float32)
    h = h + jnp.dot(emos_ref[...], w2_ref[...].T,
                    preferred_element_type=jnp.float32)
    h = jnp.maximum(h + b1_ref[...], 0.0)
    out_ref[...] = (
        jnp.dot(h, wo_ref[...].T, preferred_element_type=jnp.float32)
        + bo_ref[...]
    )


@jax.jit
def _mlp(bag_sums, emos, w1, w2, b1, wo, bo):
    nblk = B // ROWS_BLK
    return pl.pallas_call(
        _mlp_body,
        out_shape=jax.ShapeDtypeStruct((B, C), jnp.float32),
        grid=(nblk,),
        in_specs=[
            pl.BlockSpec((ROWS_BLK, D), lambda i: (i, 0)),
            pl.BlockSpec((ROWS_BLK, 2), lambda i: (i, 0)),
            pl.BlockSpec((H, D), lambda i: (0, 0)),
            pl.BlockSpec((H, 2), lambda i: (0, 0)),
            pl.BlockSpec((1, H), lambda i: (0, 0)),
            pl.BlockSpec((C, H), lambda i: (0, 0)),
            pl.BlockSpec((1, C), lambda i: (0, 0)),
        ],
        out_specs=pl.BlockSpec((ROWS_BLK, C), lambda i: (i, 0)),
    )(bag_sums, emos, w1, w2, b1, wo, bo)


def kernel(emos, tags_vec, offsets, emb_weight, hid_w, hid_b, out_w, out_b):
    del offsets
    tags2d = tags_vec.reshape(NW * CHUNKS_PER_W, CHUNK_IDX)
    bag_sums = _sc_bag_sums(emb_weight, tags2d)
    w1 = hid_w[:, :D]
    w2 = hid_w[:, D:]
    return _mlp(bag_sums, emos, w1, w2, hid_b.reshape(1, H), out_w,
                out_b.reshape(1, C))


# speedup vs baseline: 1.8150x; 1.0043x over previous
import functools

import jax
import jax.numpy as jnp
from jax import lax
from jax.experimental import pallas as pl
from jax.experimental.pallas import tpu as pltpu
from jax.experimental.pallas import tpu_sc as plsc

B = 4096
L = 50
V = 100000
D = 128
H = 256
C = 2

NC = 2
NS = 16
NW = NC * NS
BAGS_PER_W = B // NW
CHUNK_BAGS = 2
CHUNK_IDX = CHUNK_BAGS * L
CHUNKS_PER_W = BAGS_PER_W // CHUNK_BAGS
G = D // 16


IDX_PER_W = BAGS_PER_W * L
SHIFT = 4


def _sc_bag_sums_body(table_hbm, tags_hbm, tags_sh_hbm, out_hbm, idx_e, idx_o,
                      rows0, rows1, out_v, sem0, sem1):
    cid = lax.axis_index("c")
    sid = lax.axis_index("s")
    wid = sid * NC + cid
    bufs = (rows0, rows1)
    sems = (sem0, sem1)

    pltpu.sync_copy(tags_hbm.at[pl.ds(wid * IDX_PER_W, IDX_PER_W)], idx_e)
    pltpu.sync_copy(
        tags_sh_hbm.at[pl.ds(wid * IDX_PER_W, IDX_PER_W + 2 * SHIFT)], idx_o)

    def idx_slice(ci, parity):
        if parity == 0:
            off = pl.multiple_of(ci * CHUNK_IDX, 8)
            return idx_e.at[pl.ds(off, CHUNK_IDX)]
        off = pl.multiple_of(ci * CHUNK_IDX + SHIFT, 8)
        return idx_o.at[pl.ds(off, CHUNK_IDX)]

    for b in range(2):
        pltpu.async_copy(table_hbm.at[idx_slice(b, b)], bufs[b], sems[b])

    def outer(cc, carry):
        for b in range(2):
            ci = 2 * cc + b
            pltpu.make_async_copy(
                table_hbm.at[idx_slice(ci, b)], bufs[b], sems[b]).wait()
            for b2 in range(CHUNK_BAGS):
                base = b2 * L

                def accum(r, acc, _b=b, _base=base):
                    r2 = _base + r * 2
                    return tuple(
                        acc[g]
                        + bufs[_b][r2, pl.ds(g * 16, 16)]
                        + bufs[_b][r2 + 1, pl.ds(g * 16, 16)]
                        for g in range(G)
                    )

                zeros = tuple(jnp.zeros((16,), jnp.float32) for _ in range(G))
                acc = lax.fori_loop(0, L // 2, accum, zeros)
                row = CHUNK_BAGS * ci + b2
                for g in range(G):
                    out_v[row, pl.ds(g * 16, 16)] = acc[g]
            nci = ci + 2

            @pl.when(nci < CHUNKS_PER_W)
            def _():
                pltpu.async_copy(
                    table_hbm.at[idx_slice(nci, b)], bufs[b], sems[b])
        return carry

    lax.fori_loop(0, CHUNKS_PER_W // 2, outer, 0)
    pltpu.sync_copy(out_v, out_hbm.at[pl.ds(wid * BAGS_PER_W, BAGS_PER_W)])


@jax.jit
def _sc_bag_sums(emb_weight, tags_vec, tags_sh):
    mesh = plsc.VectorSubcoreMesh(core_axis_name="c", subcore_axis_name="s")
    return pl.kernel(
        _sc_bag_sums_body,
        out_type=jax.ShapeDtypeStruct((B, D), jnp.float32),
        mesh=mesh,
        scratch_types=[
            pltpu.VMEM((IDX_PER_W,), jnp.int32),
            pltpu.VMEM((IDX_PER_W + 2 * SHIFT,), jnp.int32),
            pltpu.VMEM((CHUNK_IDX, D), jnp.float32),
            pltpu.VMEM((CHUNK_IDX, D), jnp.float32),
            pltpu.VMEM((BAGS_PER_W, D), jnp.float32),
            pltpu.SemaphoreType.DMA,
            pltpu.SemaphoreType.DMA,
        ],
    )(emb_weight, tags_vec, tags_sh)


ROWS_BLK = 512


def _mlp_body(bags_ref, emos_ref, hw_ref, b1_ref, wo_ref, bo_ref, out_ref):
    feats = jnp.maximum(bags_ref[...] * (1.0 / L), 0.0)
    hw = hw_ref[...]
    h = jnp.dot(feats, hw[:, :D].T, preferred_element_type=jnp.float32)
    h = h + jnp.dot(emos_ref[...], hw[:, D:].T,
                    preferred_element_type=jnp.float32)
    h = jnp.maximum(h + b1_ref[...], 0.0)
    out_ref[...] = (
        jnp.dot(h, wo_ref[...].T, preferred_element_type=jnp.float32)
        + bo_ref[...]
    )


@jax.jit
def _mlp(bag_sums, emos, hid_w, b1, wo, bo):
    nblk = B // ROWS_BLK
    return pl.pallas_call(
        _mlp_body,
        out_shape=jax.ShapeDtypeStruct((B, C), jnp.float32),
        grid=(nblk,),
        in_specs=[
            pl.BlockSpec((ROWS_BLK, D), lambda i: (i, 0)),
            pl.BlockSpec((ROWS_BLK, 2), lambda i: (i, 0)),
            pl.BlockSpec((H, D + 2), lambda i: (0, 0)),
            pl.BlockSpec((1, H), lambda i: (0, 0)),
            pl.BlockSpec((C, H), lambda i: (0, 0)),
            pl.BlockSpec((1, C), lambda i: (0, 0)),
        ],
        out_specs=pl.BlockSpec((ROWS_BLK, C), lambda i: (i, 0)),
    )(bag_sums, emos, hid_w, b1, wo, bo)


def kernel(emos, tags_vec, offsets, emb_weight, hid_w, hid_b, out_w, out_b):
    del offsets
    tags_sh = jnp.pad(tags_vec, (SHIFT, SHIFT))
    bag_sums = _sc_bag_sums(emb_weight, tags_vec, tags_sh)
    return _mlp(bag_sums, emos, hid_w, hid_b.reshape(1, H), out_w,
                out_b.reshape(1, C))


# speedup vs baseline: 1.8732x; 1.0321x over previous
import functools

import jax
import jax.numpy as jnp
from jax import lax
from jax.experimental import pallas as pl
from jax.experimental.pallas import tpu as pltpu
from jax.experimental.pallas import tpu_sc as plsc

B = 4096
L = 50
V = 100000
D = 128
H = 256
C = 2

NC = 2
NS = 16
NW = NC * NS
BAGS_PER_W = B // NW
CHUNK_BAGS = 2
CHUNK_IDX = CHUNK_BAGS * L
CHUNKS_PER_W = BAGS_PER_W // CHUNK_BAGS
G = D // 16


IDX_PER_W = BAGS_PER_W * L
SHIFT = 4


def _sc_bag_sums_body(table_hbm, tags_hbm, tags_sh_hbm, out_hbm, idx_e, idx_o,
                      rows0, rows1, out_v, sem0, sem1):
    cid = lax.axis_index("c")
    sid = lax.axis_index("s")
    wid = sid * NC + cid
    bufs = (rows0, rows1)
    sems = (sem0, sem1)

    pltpu.sync_copy(tags_hbm.at[pl.ds(wid * IDX_PER_W, IDX_PER_W)], idx_e)
    pltpu.sync_copy(
        tags_sh_hbm.at[pl.ds(wid * IDX_PER_W, IDX_PER_W + 2 * SHIFT)], idx_o)

    def idx_slice(ci, parity):
        if parity == 0:
            off = pl.multiple_of(ci * CHUNK_IDX, 8)
            return idx_e.at[pl.ds(off, CHUNK_IDX)]
        off = pl.multiple_of(ci * CHUNK_IDX + SHIFT, 8)
        return idx_o.at[pl.ds(off, CHUNK_IDX)]

    for b in range(2):
        pltpu.async_copy(table_hbm.at[idx_slice(b, b)], bufs[b], sems[b])

    def outer(cc, carry):
        for b in range(2):
            ci = 2 * cc + b
            pltpu.make_async_copy(
                table_hbm.at[idx_slice(ci, b)], bufs[b], sems[b]).wait()
            for b2 in range(CHUNK_BAGS):
                base = b2 * L

                def accum(r, acc, _b=b, _base=base):
                    r2 = _base + r * 2
                    return tuple(
                        acc[g]
                        + bufs[_b][r2, pl.ds(g * 16, 16)]
                        + bufs[_b][r2 + 1, pl.ds(g * 16, 16)]
                        for g in range(G)
                    )

                zeros = tuple(jnp.zeros((16,), jnp.float32) for _ in range(G))
                acc = lax.fori_loop(0, L // 2, accum, zeros)
                row = CHUNK_BAGS * ci + b2
                for g in range(G):
                    out_v[row, pl.ds(g * 16, 16)] = acc[g]
            nci = ci + 2

            @pl.when(nci < CHUNKS_PER_W)
            def _():
                pltpu.async_copy(
                    table_hbm.at[idx_slice(nci, b)], bufs[b], sems[b])
        return carry

    lax.fori_loop(0, CHUNKS_PER_W // 2, outer, 0)
    pltpu.sync_copy(out_v, out_hbm.at[pl.ds(wid * BAGS_PER_W, BAGS_PER_W)])


@jax.jit
def _sc_bag_sums(emb_weight, tags_vec, tags_sh):
    mesh = plsc.VectorSubcoreMesh(core_axis_name="c", subcore_axis_name="s")
    return pl.kernel(
        _sc_bag_sums_body,
        out_type=jax.ShapeDtypeStruct((B, D), jnp.float32),
        mesh=mesh,
        scratch_types=[
            pltpu.VMEM((IDX_PER_W,), jnp.int32),
            pltpu.VMEM((IDX_PER_W + 2 * SHIFT,), jnp.int32),
            pltpu.VMEM((CHUNK_IDX, D), jnp.float32),
            pltpu.VMEM((CHUNK_IDX, D), jnp.float32),
            pltpu.VMEM((BAGS_PER_W, D), jnp.float32),
            pltpu.SemaphoreType.DMA,
            pltpu.SemaphoreType.DMA,
        ],
    )(emb_weight, tags_vec, tags_sh)


ROWS_BLK = 4096


def _mlp_body(bags_ref, emos_ref, hw_ref, b1_ref, wo_ref, bo_ref, out_ref):
    feats = jnp.maximum(bags_ref[...] * (1.0 / L), 0.0)
    hw = hw_ref[...]
    h = jnp.dot(feats, hw[:, :D].T, preferred_element_type=jnp.float32)
    h = h + jnp.dot(emos_ref[...], hw[:, D:].T,
                    preferred_element_type=jnp.float32)
    h = jnp.maximum(h + b1_ref[...], 0.0)
    out_ref[...] = (
        jnp.dot(h, wo_ref[...].T, preferred_element_type=jnp.float32)
        + bo_ref[...]
    )


@jax.jit
def _mlp(bag_sums, emos, hid_w, b1, wo, bo):
    nblk = B // ROWS_BLK
    return pl.pallas_call(
        _mlp_body,
        out_shape=jax.ShapeDtypeStruct((B, C), jnp.float32),
        grid=(nblk,),
        in_specs=[
            pl.BlockSpec((ROWS_BLK, D), lambda i: (i, 0)),
            pl.BlockSpec((ROWS_BLK, 2), lambda i: (i, 0)),
            pl.BlockSpec((H, D + 2), lambda i: (0, 0)),
            pl.BlockSpec((1, H), lambda i: (0, 0)),
            pl.BlockSpec((C, H), lambda i: (0, 0)),
            pl.BlockSpec((1, C), lambda i: (0, 0)),
        ],
        out_specs=pl.BlockSpec((ROWS_BLK, C), lambda i: (i, 0)),
    )(bag_sums, emos, hid_w, b1, wo, bo)


def kernel(emos, tags_vec, offsets, emb_weight, hid_w, hid_b, out_w, out_b):
    del offsets
    tags_sh = jnp.pad(tags_vec, (SHIFT, SHIFT))
    bag_sums = _sc_bag_sums(emb_weight, tags_vec, tags_sh)
    return _mlp(bag_sums, emos, hid_w, hid_b.reshape(1, H), out_w,
                out_b.reshape(1, C))


# speedup vs baseline: 2.3877x; 1.2747x over previous
import functools

import jax
import jax.numpy as jnp
from jax import lax
from jax.experimental import pallas as pl
from jax.experimental.pallas import tpu as pltpu
from jax.experimental.pallas import tpu_sc as plsc

B = 4096
L = 50
V = 100000
D = 128
H = 256
C = 2

NC = 2
NS = 16
NW = NC * NS
BAGS_PER_W = B // NW
CHUNK_BAGS = 2
CHUNK_IDX = CHUNK_BAGS * L
CHUNKS_PER_W = BAGS_PER_W // CHUNK_BAGS
G = D // 16


IDX_PER_W = BAGS_PER_W * L
SHIFT = 4


NBUF = 4


def _sc_bag_sums_body(table_hbm, tags_hbm, tags_sh_hbm, out_hbm, idx_e, idx_o,
                      rows0, rows1, rows2, rows3, out_v, sem0, sem1, sem2,
                      sem3):
    cid = lax.axis_index("c")
    sid = lax.axis_index("s")
    wid = sid * NC + cid
    bufs = (rows0, rows1, rows2, rows3)
    sems = (sem0, sem1, sem2, sem3)

    pltpu.sync_copy(tags_hbm.at[pl.ds(wid * IDX_PER_W, IDX_PER_W)], idx_e)
    pltpu.sync_copy(
        tags_sh_hbm.at[pl.ds(wid * IDX_PER_W, IDX_PER_W + 2 * SHIFT)], idx_o)

    def idx_slice(ci, parity):
        if parity == 0:
            off = pl.multiple_of(ci * CHUNK_IDX, 8)
            return idx_e.at[pl.ds(off, CHUNK_IDX)]
        off = pl.multiple_of(ci * CHUNK_IDX + SHIFT, 8)
        return idx_o.at[pl.ds(off, CHUNK_IDX)]

    for b in range(NBUF):
        pltpu.async_copy(table_hbm.at[idx_slice(b, b % 2)], bufs[b], sems[b])

    def outer(cc, carry):
        for b in range(NBUF):
            ci = NBUF * cc + b
            pltpu.make_async_copy(
                table_hbm.at[idx_slice(ci, b % 2)], bufs[b], sems[b]).wait()
            for b2 in range(CHUNK_BAGS):
                base = b2 * L

                def accum(r, acc, _b=b, _base=base):
                    r2 = _base + r * 2
                    return tuple(
                        acc[g]
                        + bufs[_b][r2, pl.ds(g * 16, 16)]
                        + bufs[_b][r2 + 1, pl.ds(g * 16, 16)]
                        for g in range(G)
                    )

                zeros = tuple(jnp.zeros((16,), jnp.float32) for _ in range(G))
                acc = lax.fori_loop(0, L // 2, accum, zeros)
                row = CHUNK_BAGS * ci + b2
                for g in range(G):
                    out_v[row, pl.ds(g * 16, 16)] = acc[g]
            nci = ci + NBUF

            @pl.when(nci < CHUNKS_PER_W)
            def _():
                pltpu.async_copy(
                    table_hbm.at[idx_slice(nci, b % 2)], bufs[b], sems[b])
        return carry

    lax.fori_loop(0, CHUNKS_PER_W // NBUF, outer, 0)
    pltpu.sync_copy(out_v, out_hbm.at[pl.ds(wid * BAGS_PER_W, BAGS_PER_W)])


@jax.jit
def _sc_bag_sums(emb_weight, tags_vec, tags_sh):
    mesh = plsc.VectorSubcoreMesh(core_axis_name="c", subcore_axis_name="s")
    return pl.kernel(
        _sc_bag_sums_body,
        out_type=jax.ShapeDtypeStruct((B, D), jnp.float32),
        mesh=mesh,
        scratch_types=[
            pltpu.VMEM((IDX_PER_W,), jnp.int32),
            pltpu.VMEM((IDX_PER_W + 2 * SHIFT,), jnp.int32),
            pltpu.VMEM((CHUNK_IDX, D), jnp.float32),
            pltpu.VMEM((CHUNK_IDX, D), jnp.float32),
            pltpu.VMEM((CHUNK_IDX, D), jnp.float32),
            pltpu.VMEM((CHUNK_IDX, D), jnp.float32),
            pltpu.VMEM((BAGS_PER_W, D), jnp.float32),
            pltpu.SemaphoreType.DMA,
            pltpu.SemaphoreType.DMA,
            pltpu.SemaphoreType.DMA,
            pltpu.SemaphoreType.DMA,
        ],
    )(emb_weight, tags_vec, tags_sh)


ROWS_BLK = 4096


def _mlp_body(bags_ref, emos_ref, hw_ref, b1_ref, wo_ref, bo_ref, out_ref):
    feats = jnp.maximum(bags_ref[...] * (1.0 / L), 0.0)
    hw = hw_ref[...]
    h = jnp.dot(feats, hw[:, :D].T, preferred_element_type=jnp.float32)
    h = h + jnp.dot(emos_ref[...], hw[:, D:].T,
                    preferred_element_type=jnp.float32)
    h = jnp.maximum(h + b1_ref[...], 0.0)
    out_ref[...] = (
        jnp.dot(h, wo_ref[...].T, preferred_element_type=jnp.float32)
        + bo_ref[...]
    )


@jax.jit
def _mlp(bag_sums, emos, hid_w, b1, wo, bo):
    nblk = B // ROWS_BLK
    return pl.pallas_call(
        _mlp_body,
        out_shape=jax.ShapeDtypeStruct((B, C), jnp.float32),
        grid=(nblk,),
        in_specs=[
            pl.BlockSpec((ROWS_BLK, D), lambda i: (i, 0)),
            pl.BlockSpec((ROWS_BLK, 2), lambda i: (i, 0)),
            pl.BlockSpec((H, D + 2), lambda i: (0, 0)),
            pl.BlockSpec((1, H), lambda i: (0, 0)),
            pl.BlockSpec((C, H), lambda i: (0, 0)),
            pl.BlockSpec((1, C), lambda i: (0, 0)),
        ],
        out_specs=pl.BlockSpec((ROWS_BLK, C), lambda i: (i, 0)),
    )(bag_sums, emos, hid_w, b1, wo, bo)


def kernel(emos, tags_vec, offsets, emb_weight, hid_w, hid_b, out_w, out_b):
    del offsets
    tags_sh = jnp.pad(tags_vec, (SHIFT, SHIFT))
    bag_sums = _sc_bag_sums(emb_weight, tags_vec, tags_sh)
    return _mlp(bag_sums, emos, hid_w, hid_b.reshape(1, H), out_w,
                out_b.reshape(1, C))
